# speedup vs baseline: 7.8276x; 7.8276x over previous
import jax
import jax.numpy as jnp
from jax.experimental import pallas as pl
from jax.experimental.pallas import tpu as pltpu

_STATIC = 26
_CONT = 13
_ED = 64
_BATCH = 16384
_R = 256


def _tc_body(si_ref, ci_ref, tab2_ref, wt_ref, b_ref, out_ref):
    si = si_ref[...]
    tab2 = tab2_ref[...]
    lane = jax.lax.broadcasted_iota(jnp.int32, (_R, 2 * _ED), 1)
    for p in range(_STATIC // 2):
        c0 = si[:, 2 * p: 2 * p + 1]
        c1 = si[:, 2 * p + 1: 2 * p + 2]
        sel = jnp.where(lane < 32, c0, c1 + 32)
        onehot = (sel == lane).astype(jnp.float32)
        out_ref[:, 2 * _ED * p: 2 * _ED * (p + 1)] = jnp.dot(
            onehot, tab2, preferred_element_type=jnp.float32)
    cont = ci_ref[...]
    out_ref[:, _STATIC * _ED:] = (
        jnp.dot(cont, wt_ref[...], preferred_element_type=jnp.float32) + b_ref[...])


def kernel(static_input, continuous_input, table, W, b):
    tab2 = jnp.zeros((2 * _ED, 2 * _ED), jnp.float32)
    tab2 = tab2.at[0:_STATIC, 0:_ED].set(table)
    tab2 = tab2.at[32:32 + _STATIC, _ED:2 * _ED].set(table)
    wt = W.T
    b2 = b.reshape(1, _CONT * _ED)

    grid = (_BATCH // _R,)
    out = pl.pallas_call(
        _tc_body,
        grid=grid,
        in_specs=[
            pl.BlockSpec((_R, _STATIC), lambda i: (i, 0)),
            pl.BlockSpec((_R, _CONT), lambda i: (i, 0)),
            pl.BlockSpec((2 * _ED, 2 * _ED), lambda i: (0, 0)),
            pl.BlockSpec((_CONT, _CONT * _ED), lambda i: (0, 0)),
            pl.BlockSpec((1, _CONT * _ED), lambda i: (0, 0)),
        ],
        out_specs=pl.BlockSpec((_R, (_STATIC + _CONT) * _ED), lambda i: (i, 0)),
        out_shape=jax.ShapeDtypeStruct((_BATCH, (_STATIC + _CONT) * _ED), jnp.float32),
    )(static_input, continuous_input, tab2, wt, b2)
    return out
